# Initial kernel scaffold; baseline (speedup 1.0000x reference)
import jax
import jax.numpy as jnp
from jax.experimental import pallas as pl


def kernel(node_feats, edge_index, edge_attr, np_W, np_b, ep_W, ep_b, g1_W, g1_as, g1_ad, g1_b, g2_W, g2_as, g2_ad, g2_b, op_W, op_b):
    raise NotImplementedError("write your pallas kernel here")


# speedup vs baseline: 26.3595x; 26.3595x over previous
import functools

import jax
import jax.numpy as jnp
from jax import lax
from jax.experimental import pallas as pl
from jax.experimental.pallas import tpu as pltpu
from jax.experimental.pallas import tpu_sc as plsc

N = 10000
E = 160000
NC = 2
NS = 16
NPAD = 10240
ROWS_PER_TILE = NPAD // NS

B = 128
ETOT = E + N
NBLK_C = -(-ETOT // (NS * B))
PT_C = NBLK_C * B
EPAD_C = NS * PT_C
NBLK_B = -(-E // (NC * NS * B))
PT_B = NBLK_B * B
EPAD_B = NC * NS * PT_B

TPAD = 4 * (N + 16)
DLEN = 2 * NPAD

_mesh = plsc.VectorSubcoreMesh(
    core_axis_name="c", subcore_axis_name="s", num_cores=NC, num_subcores=NS)


def _f32(shape):
  return jax.ShapeDtypeStruct(shape, jnp.float32)


def _writeout(c, s, src_ref, dst_ref):
  @pl.when(s < NS - 1)
  def _():
    o = s * ROWS_PER_TILE
    pltpu.sync_copy(src_ref.at[pl.ds(o, ROWS_PER_TILE)],
                    dst_ref.at[c, pl.ds(o, ROWS_PER_TILE)])
  @pl.when(s == NS - 1)
  def _():
    o = (NS - 1) * ROWS_PER_TILE
    pltpu.sync_copy(src_ref.at[pl.ds(o, N - o)],
                    dst_ref.at[c, pl.ds(o, N - o)])


@functools.partial(
    pl.kernel,
    out_type=_f32((NC, N, 128)),
    mesh=_mesh,
    scratch_types=[
        pltpu.VMEM((B, 16), jnp.float32),
        pltpu.VMEM((B,), jnp.int32),
        pltpu.VMEM((B, 128), jnp.float32),
        pltpu.VMEM_SHARED((NPAD, 128), jnp.float32),
        pltpu.SemaphoreType.DMA,
    ],
    compiler_params=pltpu.CompilerParams(needs_layout_passes=False),
)
def _sc_scatter_mean(ea_hbm, dst_hbm, z128_hbm, eacc_o,
                     abuf, dbuf, ebuf, acc, sem):
  c = lax.axis_index("c")
  s = lax.axis_index("s")
  pltpu.sync_copy(z128_hbm, acc.at[pl.ds(s * ROWS_PER_TILE, ROWS_PER_TILE)])
  io16 = lax.iota(jnp.int32, 16)
  one16 = jnp.where(io16 == 0, 1.0, 0.0)
  z16 = jnp.zeros((16,), jnp.float32)

  @plsc.parallel_loop(0, B, 1, unroll=8)
  def _init(e):
    ebuf[e, pl.ds(16, 16)] = one16
    for r in range(2, 8):
      ebuf[e, pl.ds(r * 16, 16)] = z16

  plsc.subcore_barrier()
  base0 = c * (NS * PT_B) + s * PT_B

  def blk(b, carry):
    base = base0 + b * B
    pltpu.sync_copy(dst_hbm.at[pl.ds(base, B)], dbuf)
    pltpu.sync_copy(ea_hbm.at[pl.ds(base, B)], abuf)

    @plsc.parallel_loop(0, B, 1, unroll=8)
    def _fill(e):
      ebuf[e, pl.ds(0, 16)] = abuf[e, pl.ds(0, 16)]

    pltpu.async_copy(ebuf, acc.at[dbuf], sem, add=True).wait()
    return carry

  lax.fori_loop(0, NBLK_B, blk, 0)
  plsc.subcore_barrier()
  _writeout(c, s, acc, eacc_o)


@functools.partial(
    pl.kernel,
    out_type=(_f32((NC, EPAD_C)), _f32((NC, EPAD_C)), _f32((NC, NS, DLEN))),
    mesh=_mesh,
    scratch_types=[
        pltpu.VMEM((TPAD,), jnp.float32),
        pltpu.VMEM((B,), jnp.int32),
        pltpu.VMEM((B,), jnp.int32),
        pltpu.VMEM((B,), jnp.float32),
        pltpu.VMEM((B,), jnp.float32),
        pltpu.VMEM((DLEN,), jnp.float32),
    ],
    compiler_params=pltpu.CompilerParams(needs_layout_passes=False),
)
def _sc_gat_alpha(t_hbm, src_hbm, dst_hbm, zflat_hbm,
                  w0_o, w1_o, den_o, tv, sbuf, dbuf, w0b, w1b, den):
  c = lax.axis_index("c")
  s = lax.axis_index("s")
  pltpu.sync_copy(t_hbm.at[c], tv)
  pltpu.sync_copy(zflat_hbm, den)

  base0 = s * PT_C

  def blk(b, carry):
    base = base0 + b * B
    pltpu.sync_copy(src_hbm.at[pl.ds(base, B)], sbuf)
    pltpu.sync_copy(dst_hbm.at[pl.ds(base, B)], dbuf)
    for j in range(B // 16):
      sv = sbuf[pl.ds(j * 16, 16)]
      dv = dbuf[pl.ds(j * 16, 16)]
      s4 = sv * 4
      d4 = dv * 4
      a0 = plsc.load_gather(tv, [s4]) + plsc.load_gather(tv, [d4 + 2])
      a1 = plsc.load_gather(tv, [s4 + 1]) + plsc.load_gather(tv, [d4 + 3])
      a0 = jnp.where(a0 >= 0.0, a0, 0.2 * a0)
      a1 = jnp.where(a1 >= 0.0, a1, 0.2 * a1)
      w0 = jnp.exp(a0)
      w1 = jnp.exp(a1)
      w0b[pl.ds(j * 16, 16)] = w0
      w1b[pl.ds(j * 16, 16)] = w1
      d2 = dv * 2
      plsc.addupdate_scatter(den, [d2], w0)
      plsc.addupdate_scatter(den, [d2 + 1], w1)
    pltpu.sync_copy(w0b, w0_o.at[c, pl.ds(base, B)])
    pltpu.sync_copy(w1b, w1_o.at[c, pl.ds(base, B)])
    return carry

  lax.fori_loop(0, NBLK_C, blk, 0)
  pltpu.sync_copy(den, den_o.at[c, s])


@functools.partial(
    pl.kernel,
    out_type=_f32((NC, N, 128)),
    mesh=_mesh,
    scratch_types=[
        pltpu.VMEM((B,), jnp.int32),
        pltpu.VMEM((B,), jnp.int32),
        pltpu.VMEM((B,), jnp.int32),
        pltpu.VMEM((B,), jnp.float32),
        pltpu.VMEM((B,), jnp.float32),
        pltpu.VMEM((B, 128), jnp.float32),
        pltpu.VMEM_SHARED((NPAD, 128), jnp.float32),
        pltpu.SemaphoreType.DMA,
        pltpu.SemaphoreType.DMA,
    ],
    compiler_params=pltpu.CompilerParams(needs_layout_passes=False),
)
def _sc_gat_aggregate(h_hbm, w0_hbm, w1_hbm, src_hbm, dst_hbm, z128_hbm,
                      agg_o, sbuf, dbuf, gidx, w0b, w1b, gbuf, acc, sem,
                      sem2):
  c = lax.axis_index("c")
  s = lax.axis_index("s")
  pltpu.sync_copy(z128_hbm, acc.at[pl.ds(s * ROWS_PER_TILE, ROWS_PER_TILE)])
  plsc.subcore_barrier()

  base0 = s * PT_C
  row0 = c * N

  def blk(b, carry):
    base = base0 + b * B
    pltpu.sync_copy(src_hbm.at[pl.ds(base, B)], sbuf)
    pltpu.sync_copy(dst_hbm.at[pl.ds(base, B)], dbuf)
    pltpu.sync_copy(w0_hbm.at[c, pl.ds(base, B)], w0b)
    pltpu.sync_copy(w1_hbm.at[c, pl.ds(base, B)], w1b)
    for j in range(B // 16):
      gidx[pl.ds(j * 16, 16)] = sbuf[pl.ds(j * 16, 16)] + row0
    pltpu.async_copy(h_hbm.at[gidx], gbuf, sem).wait()

    @plsc.parallel_loop(0, B, 1, unroll=8)
    def _mul(e):
      ev = jnp.full((16,), e, dtype=jnp.int32)
      s0 = plsc.load_gather(w0b, [ev])
      s1 = plsc.load_gather(w1b, [ev])
      for r in range(4):
        gbuf[e, pl.ds(r * 16, 16)] = gbuf[e, pl.ds(r * 16, 16)] * s0
      for r in range(4, 8):
        gbuf[e, pl.ds(r * 16, 16)] = gbuf[e, pl.ds(r * 16, 16)] * s1

    pltpu.async_copy(gbuf, acc.at[dbuf], sem2, add=True).wait()
    return carry

  lax.fori_loop(0, NBLK_C, blk, 0)
  plsc.subcore_barrier()
  _writeout(c, s, acc, agg_o)


TCM = 400
_GRID = N // TCM


def _elu(x):
  return jnp.where(x > 0.0, x, jnp.exp(jnp.minimum(x, 0.0)) - 1.0)


def _split_heads(h, a_s, a_d, h_o, t_o):
  h_o[0] = h[:, 0:128]
  h_o[1] = h[:, 128:256]
  t_o[0] = jnp.concatenate([a_s[:, 0:2], a_d[:, 0:2]], axis=1)
  t_o[1] = jnp.concatenate([a_s[:, 2:4], a_d[:, 2:4]], axis=1)


def _tc1_body(nf, eacc, npW, npb, epW, epb, g1W, asr, adr, S, h_o, t_o):
  x1 = jnp.dot(nf[:], npW[:], preferred_element_type=jnp.float32) + npb[:]
  es = eacc[0, :, 0:16] + eacc[1, :, 0:16]
  cn = jnp.maximum(eacc[0, :, 16:17] + eacc[1, :, 16:17], 1.0)
  ne = jnp.dot(es, epW[:], preferred_element_type=jnp.float32) / cn + epb[:]
  h = (jnp.dot(x1, g1W[0:256, :], preferred_element_type=jnp.float32)
       + jnp.dot(ne, g1W[256:512, :], preferred_element_type=jnp.float32))
  a_s = jnp.dot(h * asr[:], S[:], preferred_element_type=jnp.float32)
  a_d = jnp.dot(h * adr[:], S[:], preferred_element_type=jnp.float32)
  _split_heads(h, a_s, a_d, h_o, t_o)


def _normalize(agg, den, b_row):
  dsum = jnp.sum(den[:], axis=1)
  d0 = dsum[0, :, 0:1] + 1e-16
  d1 = dsum[0, :, 1:2] + 1e-16
  d2 = dsum[1, :, 0:1] + 1e-16
  d3 = dsum[1, :, 1:2] + 1e-16
  x = jnp.concatenate(
      [agg[0, :, 0:64] / d0, agg[0, :, 64:128] / d1,
       agg[1, :, 0:64] / d2, agg[1, :, 64:128] / d3], axis=1)
  return _elu(x + b_row[:])


def _tc2_body(agg, den, g1b, g2W, asr, adr, S, h_o, t_o):
  x = _normalize(agg, den, g1b)
  h = jnp.dot(x, g2W[:], preferred_element_type=jnp.float32)
  a_s = jnp.dot(h * asr[:], S[:], preferred_element_type=jnp.float32)
  a_d = jnp.dot(h * adr[:], S[:], preferred_element_type=jnp.float32)
  _split_heads(h, a_s, a_d, h_o, t_o)


def _tc3_body(agg, den, g2b, opW, opb, out_o):
  x = _normalize(agg, den, g2b)
  out_o[:] = jnp.dot(x, opW[:], preferred_element_type=jnp.float32) + opb[:]


def _rowspec(width):
  return pl.BlockSpec((TCM, width), lambda i: (i, 0))


def _corespec(width):
  return pl.BlockSpec((NC, TCM, width), lambda i: (0, i, 0))


def _denspec():
  return pl.BlockSpec((NC, NS, TCM, 2), lambda i: (0, 0, i, 0))


def _full(shape):
  return pl.BlockSpec(shape, lambda i: tuple(0 for _ in shape))


_tc1 = pl.pallas_call(
    _tc1_body,
    grid=(_GRID,),
    in_specs=[
        _rowspec(256), _corespec(128),
        _full((256, 256)), _full((1, 256)), _full((16, 256)), _full((1, 256)),
        _full((512, 256)), _full((1, 256)), _full((1, 256)), _full((256, 4)),
    ],
    out_specs=[_corespec(128), _corespec(4)],
    out_shape=[_f32((NC, N, 128)), _f32((NC, N, 4))],
)

_tc2 = pl.pallas_call(
    _tc2_body,
    grid=(_GRID,),
    in_specs=[
        _corespec(128), _denspec(), _full((1, 256)), _full((256, 256)),
        _full((1, 256)), _full((1, 256)), _full((256, 4)),
    ],
    out_specs=[_corespec(128), _corespec(4)],
    out_shape=[_f32((NC, N, 128)), _f32((NC, N, 4))],
)

_tc3 = pl.pallas_call(
    _tc3_body,
    grid=(_GRID,),
    in_specs=[
        _corespec(128), _denspec(), _full((1, 256)), _full((256, 256)),
        _full((1, 256)),
    ],
    out_specs=_rowspec(256),
    out_shape=_f32((N, 256)),
)


def kernel(node_feats, edge_index, edge_attr, np_W, np_b, ep_W, ep_b,
           g1_W, g1_as, g1_ad, g1_b, g2_W, g2_as, g2_ad, g2_b, op_W, op_b):
  i32 = jnp.int32
  loop = jnp.arange(N, dtype=i32)
  src_c = jnp.concatenate(
      [edge_index[0].astype(i32), loop,
       jnp.zeros((EPAD_C - ETOT,), dtype=i32)])
  dst_c = jnp.concatenate(
      [edge_index[1].astype(i32), loop,
       jnp.full((EPAD_C - ETOT,), N, dtype=i32)])
  dst_b = jnp.concatenate(
      [edge_index[1].astype(i32), jnp.full((EPAD_B - E,), N, dtype=i32)])
  ea_pad = jnp.concatenate(
      [edge_attr, jnp.zeros((EPAD_B - E, 16), dtype=jnp.float32)])

  z128 = jnp.zeros((ROWS_PER_TILE, 128), jnp.float32)
  zflat = jnp.zeros((DLEN,), jnp.float32)
  S = (jnp.arange(256)[:, None] // 64 == jnp.arange(4)[None, :]
       ).astype(jnp.float32)

  eacc = _sc_scatter_mean(ea_pad, dst_b, z128)

  h1, t1 = _tc1(node_feats, eacc, np_W, np_b.reshape(1, 256), ep_W,
                ep_b.reshape(1, 256), g1_W, g1_as.reshape(1, 256),
                g1_ad.reshape(1, 256), S)
  t1p = jnp.pad(t1.reshape(NC, 4 * N), ((0, 0), (0, TPAD - 4 * N)))
  w0_1, w1_1, den1 = _sc_gat_alpha(t1p, src_c, dst_c, zflat)
  agg1 = _sc_gat_aggregate(h1.reshape(NC * N, 128), w0_1, w1_1, src_c, dst_c,
                           z128)

  h2, t2 = _tc2(agg1, den1.reshape(NC, NS, NPAD, 2), g1_b.reshape(1, 256),
                g2_W, g2_as.reshape(1, 256), g2_ad.reshape(1, 256), S)
  t2p = jnp.pad(t2.reshape(NC, 4 * N), ((0, 0), (0, TPAD - 4 * N)))
  w0_2, w1_2, den2 = _sc_gat_alpha(t2p, src_c, dst_c, zflat)
  agg2 = _sc_gat_aggregate(h2.reshape(NC * N, 128), w0_2, w1_2, src_c, dst_c,
                           z128)

  return _tc3(agg2, den2.reshape(NC, NS, NPAD, 2), g2_b.reshape(1, 256),
              op_W, op_b.reshape(1, 256))


# speedup vs baseline: 45.2383x; 1.7162x over previous
import functools

import jax
import jax.numpy as jnp
from jax import lax
from jax.experimental import pallas as pl
from jax.experimental.pallas import tpu as pltpu
from jax.experimental.pallas import tpu_sc as plsc

N = 10000
E = 160000
NC = 2
NS = 16
NPAD = 10240
ROWS_PER_TILE = NPAD // NS

B = 128
ETOT = E + N
NBLK_C = -(-ETOT // (NS * B))
PT_C = NBLK_C * B
EPAD_C = NS * PT_C
NBLK_B = -(-E // (NC * NS * B))
PT_B = NBLK_B * B
EPAD_B = NC * NS * PT_B

TPAD = 4 * (N + 16)
DLEN = 2 * NPAD

_mesh = plsc.VectorSubcoreMesh(
    core_axis_name="c", subcore_axis_name="s", num_cores=NC, num_subcores=NS)


def _f32(shape):
  return jax.ShapeDtypeStruct(shape, jnp.float32)


def _writeout(c, s, src_ref, dst_ref):
  @pl.when(s < NS - 1)
  def _():
    o = s * ROWS_PER_TILE
    pltpu.sync_copy(src_ref.at[pl.ds(o, ROWS_PER_TILE)],
                    dst_ref.at[c, pl.ds(o, ROWS_PER_TILE)])
  @pl.when(s == NS - 1)
  def _():
    o = (NS - 1) * ROWS_PER_TILE
    pltpu.sync_copy(src_ref.at[pl.ds(o, N - o)],
                    dst_ref.at[c, pl.ds(o, N - o)])


@functools.partial(
    pl.kernel,
    out_type=_f32((NC, N, 128)),
    mesh=_mesh,
    scratch_types=[
        pltpu.VMEM((2, 16 * B), jnp.float32),
        pltpu.VMEM((NBLK_B, B), jnp.int32),
        pltpu.VMEM((2, B, 128), jnp.float32),
        pltpu.VMEM_SHARED((NPAD, 128), jnp.float32),
        pltpu.SemaphoreType.DMA,
        pltpu.SemaphoreType.DMA,
    ],
    compiler_params=pltpu.CompilerParams(needs_layout_passes=False),
)
def _sc_scatter_mean(ea_hbm, dst2_hbm, z128_hbm, eacc_o,
                     abuf, dbuf, ebuf, acc, psem, ssem):
  c = lax.axis_index("c")
  s = lax.axis_index("s")
  tile = c * NS + s
  pltpu.sync_copy(z128_hbm, acc.at[pl.ds(s * ROWS_PER_TILE, ROWS_PER_TILE)])
  pltpu.sync_copy(dst2_hbm.at[pl.ds(tile * NBLK_B, NBLK_B)], dbuf)
  io16 = lax.iota(jnp.int32, 16)
  one16 = jnp.where(io16 == 0, 1.0, 0.0)
  z16 = jnp.zeros((16,), jnp.float32)

  @plsc.parallel_loop(0, B, 1, unroll=8)
  def _init(e):
    for sl in range(2):
      ebuf[sl, e, pl.ds(16, 16)] = one16
      for r in range(2, 8):
        ebuf[sl, e, pl.ds(r * 16, 16)] = z16

  plsc.subcore_barrier()
  ebase = c * (NS * PT_B) + s * PT_B

  def _load(b, slot):
    pltpu.async_copy(
        ea_hbm.at[pl.ds((ebase + b * B) * 16, 16 * B)], abuf.at[slot], psem)

  def _wait_load(b, slot):
    pltpu.make_async_copy(
        ea_hbm.at[pl.ds((ebase + b * B) * 16, 16 * B)], abuf.at[slot],
        psem).wait()

  def _wait_scatter(slot):
    pltpu.make_async_copy(ebuf.at[slot], acc.at[dbuf.at[0]], ssem).wait()

  _load(0, 0)

  def blk(b, carry):
    slot = lax.rem(b, 2)
    nslot = 1 - slot

    @pl.when(b >= 1)
    def _():
      _wait_scatter(nslot)

    @pl.when(b + 1 < NBLK_B)
    def _():
      _load(b + 1, nslot)

    _wait_load(b, slot)

    @plsc.parallel_loop(0, B, 1, unroll=8)
    def _fill(e):
      ebuf[slot, e, pl.ds(0, 16)] = abuf[slot, pl.ds(e * 16, 16)]

    pltpu.async_copy(ebuf.at[slot], acc.at[dbuf.at[b]], ssem, add=True)
    return carry

  lax.fori_loop(0, NBLK_B, blk, 0)
  _wait_scatter(lax.rem(NBLK_B - 1, 2))
  plsc.subcore_barrier()
  _writeout(c, s, acc, eacc_o)


@functools.partial(
    pl.kernel,
    out_type=(_f32((NC, EPAD_C)), _f32((NC, EPAD_C)), _f32((NC, NS, DLEN))),
    mesh=_mesh,
    scratch_types=[
        pltpu.VMEM((TPAD,), jnp.float32),
        pltpu.VMEM((PT_C,), jnp.int32),
        pltpu.VMEM((PT_C,), jnp.int32),
        pltpu.VMEM((PT_C,), jnp.float32),
        pltpu.VMEM((PT_C,), jnp.float32),
        pltpu.VMEM((DLEN,), jnp.float32),
    ],
    compiler_params=pltpu.CompilerParams(needs_layout_passes=False),
)
def _sc_gat_alpha(t_hbm, src_hbm, dst_hbm, zflat_hbm,
                  w0_o, w1_o, den_o, tv, sbuf, dbuf, w0b, w1b, den):
  c = lax.axis_index("c")
  s = lax.axis_index("s")
  base0 = s * PT_C
  pltpu.sync_copy(t_hbm.at[c], tv)
  pltpu.sync_copy(zflat_hbm, den)
  pltpu.sync_copy(src_hbm.at[pl.ds(base0, PT_C)], sbuf)
  pltpu.sync_copy(dst_hbm.at[pl.ds(base0, PT_C)], dbuf)

  def grp(j, carry):
    o = pl.multiple_of(j * 16, 16)
    sv = sbuf[pl.ds(o, 16)]
    dv = dbuf[pl.ds(o, 16)]
    s4 = sv * 4
    d4 = dv * 4
    a0 = plsc.load_gather(tv, [s4]) + plsc.load_gather(tv, [d4 + 2])
    a1 = plsc.load_gather(tv, [s4 + 1]) + plsc.load_gather(tv, [d4 + 3])
    a0 = jnp.where(a0 >= 0.0, a0, 0.2 * a0)
    a1 = jnp.where(a1 >= 0.0, a1, 0.2 * a1)
    w0 = jnp.exp(a0)
    w1 = jnp.exp(a1)
    w0b[pl.ds(o, 16)] = w0
    w1b[pl.ds(o, 16)] = w1
    d2 = dv * 2
    plsc.addupdate_scatter(den, [d2], w0)
    plsc.addupdate_scatter(den, [d2 + 1], w1)
    return carry

  lax.fori_loop(0, PT_C // 16, grp, 0)
  pltpu.sync_copy(w0b, w0_o.at[c, pl.ds(base0, PT_C)])
  pltpu.sync_copy(w1b, w1_o.at[c, pl.ds(base0, PT_C)])
  pltpu.sync_copy(den, den_o.at[c, s])


@functools.partial(
    pl.kernel,
    out_type=_f32((NC, N, 128)),
    mesh=_mesh,
    scratch_types=[
        pltpu.VMEM((PT_C,), jnp.int32),
        pltpu.VMEM((2, B), jnp.int32),
        pltpu.VMEM((2, B), jnp.float32),
        pltpu.VMEM((2, B), jnp.float32),
        pltpu.VMEM((2, B, 128), jnp.float32),
        pltpu.VMEM_SHARED((NPAD, 128), jnp.float32),
        pltpu.SemaphoreType.DMA,
        pltpu.SemaphoreType.DMA,
        pltpu.SemaphoreType.DMA,
    ],
    compiler_params=pltpu.CompilerParams(needs_layout_passes=False),
)
def _sc_gat_aggregate(h_hbm, gsrc_hbm, w0_hbm, w1_hbm, dst_hbm, z128_hbm,
                      agg_o, gidxf, dbuf, w0b, w1b, gbuf, acc, gsem, ssem,
                      psem):
  c = lax.axis_index("c")
  s = lax.axis_index("s")
  base0 = s * PT_C
  pltpu.sync_copy(z128_hbm, acc.at[pl.ds(s * ROWS_PER_TILE, ROWS_PER_TILE)])
  pltpu.sync_copy(gsrc_hbm.at[c, pl.ds(base0, PT_C)], gidxf)
  plsc.subcore_barrier()

  def _loads(b, slot):
    base = base0 + b * B
    pltpu.async_copy(dst_hbm.at[pl.ds(base, B)], dbuf.at[slot], psem)
    pltpu.async_copy(w0_hbm.at[c, pl.ds(base, B)], w0b.at[slot], psem)
    pltpu.async_copy(w1_hbm.at[c, pl.ds(base, B)], w1b.at[slot], psem)
    pltpu.async_copy(h_hbm.at[gidxf.at[pl.ds(b * B, B)]], gbuf.at[slot],
                     gsem)

  def _wait_loads(b, slot):
    base = base0 + b * B
    pltpu.make_async_copy(dst_hbm.at[pl.ds(base, B)], dbuf.at[slot],
                          psem).wait()
    pltpu.make_async_copy(w0_hbm.at[c, pl.ds(base, B)], w0b.at[slot],
                          psem).wait()
    pltpu.make_async_copy(w1_hbm.at[c, pl.ds(base, B)], w1b.at[slot],
                          psem).wait()
    pltpu.make_async_copy(h_hbm.at[gidxf.at[pl.ds(b * B, B)]], gbuf.at[slot],
                          gsem).wait()

  def _wait_scatter(slot):
    pltpu.make_async_copy(gbuf.at[slot], acc.at[dbuf.at[slot]], ssem).wait()

  _loads(0, 0)

  def blk(b, carry):
    slot = lax.rem(b, 2)
    nslot = 1 - slot

    @pl.when(b >= 1)
    def _():
      _wait_scatter(nslot)

    @pl.when(b + 1 < NBLK_C)
    def _():
      _loads(b + 1, nslot)

    _wait_loads(b, slot)

    @plsc.parallel_loop(0, B, 1, unroll=8)
    def _mul(e):
      ev = jnp.full((16,), e, dtype=jnp.int32)
      s0 = plsc.load_gather(w0b.at[slot], [ev])
      s1 = plsc.load_gather(w1b.at[slot], [ev])
      for r in range(4):
        gbuf[slot, e, pl.ds(r * 16, 16)] = gbuf[slot, e, pl.ds(r * 16, 16)] * s0
      for r in range(4, 8):
        gbuf[slot, e, pl.ds(r * 16, 16)] = gbuf[slot, e, pl.ds(r * 16, 16)] * s1

    pltpu.async_copy(gbuf.at[slot], acc.at[dbuf.at[slot]], ssem, add=True)
    return carry

  lax.fori_loop(0, NBLK_C, blk, 0)
  _wait_scatter(lax.rem(NBLK_C - 1, 2))
  plsc.subcore_barrier()
  _writeout(c, s, acc, agg_o)


TCM = 400
_GRID = N // TCM


def _elu(x):
  return jnp.where(x > 0.0, x, jnp.exp(jnp.minimum(x, 0.0)) - 1.0)


def _split_heads(h, a_s, a_d, h_o, t_o):
  h_o[0] = h[:, 0:128]
  h_o[1] = h[:, 128:256]
  t_o[0] = jnp.concatenate([a_s[:, 0:2], a_d[:, 0:2]], axis=1)
  t_o[1] = jnp.concatenate([a_s[:, 2:4], a_d[:, 2:4]], axis=1)


def _tc1_body(nf, eacc, npW, npb, epW, epb, g1W, asr, adr, S, h_o, t_o):
  x1 = jnp.dot(nf[:], npW[:], preferred_element_type=jnp.float32) + npb[:]
  es = eacc[0, :, 0:16] + eacc[1, :, 0:16]
  cn = jnp.maximum(eacc[0, :, 16:17] + eacc[1, :, 16:17], 1.0)
  ne = jnp.dot(es, epW[:], preferred_element_type=jnp.float32) / cn + epb[:]
  h = (jnp.dot(x1, g1W[0:256, :], preferred_element_type=jnp.float32)
       + jnp.dot(ne, g1W[256:512, :], preferred_element_type=jnp.float32))
  a_s = jnp.dot(h * asr[:], S[:], preferred_element_type=jnp.float32)
  a_d = jnp.dot(h * adr[:], S[:], preferred_element_type=jnp.float32)
  _split_heads(h, a_s, a_d, h_o, t_o)


def _normalize(agg, den, b_row):
  dsum = jnp.sum(den[:], axis=1)
  d0 = dsum[0, :, 0:1] + 1e-16
  d1 = dsum[0, :, 1:2] + 1e-16
  d2 = dsum[1, :, 0:1] + 1e-16
  d3 = dsum[1, :, 1:2] + 1e-16
  x = jnp.concatenate(
      [agg[0, :, 0:64] / d0, agg[0, :, 64:128] / d1,
       agg[1, :, 0:64] / d2, agg[1, :, 64:128] / d3], axis=1)
  return _elu(x + b_row[:])


def _tc2_body(agg, den, g1b, g2W, asr, adr, S, h_o, t_o):
  x = _normalize(agg, den, g1b)
  h = jnp.dot(x, g2W[:], preferred_element_type=jnp.float32)
  a_s = jnp.dot(h * asr[:], S[:], preferred_element_type=jnp.float32)
  a_d = jnp.dot(h * adr[:], S[:], preferred_element_type=jnp.float32)
  _split_heads(h, a_s, a_d, h_o, t_o)


def _tc3_body(agg, den, g2b, opW, opb, out_o):
  x = _normalize(agg, den, g2b)
  out_o[:] = jnp.dot(x, opW[:], preferred_element_type=jnp.float32) + opb[:]


def _rowspec(width):
  return pl.BlockSpec((TCM, width), lambda i: (i, 0))


def _corespec(width):
  return pl.BlockSpec((NC, TCM, width), lambda i: (0, i, 0))


def _denspec():
  return pl.BlockSpec((NC, NS, TCM, 2), lambda i: (0, 0, i, 0))


def _full(shape):
  return pl.BlockSpec(shape, lambda i: tuple(0 for _ in shape))


_tc1 = pl.pallas_call(
    _tc1_body,
    grid=(_GRID,),
    in_specs=[
        _rowspec(256), _corespec(128),
        _full((256, 256)), _full((1, 256)), _full((16, 256)), _full((1, 256)),
        _full((512, 256)), _full((1, 256)), _full((1, 256)), _full((256, 4)),
    ],
    out_specs=[_corespec(128), _corespec(4)],
    out_shape=[_f32((NC, N, 128)), _f32((NC, N, 4))],
)

_tc2 = pl.pallas_call(
    _tc2_body,
    grid=(_GRID,),
    in_specs=[
        _corespec(128), _denspec(), _full((1, 256)), _full((256, 256)),
        _full((1, 256)), _full((1, 256)), _full((256, 4)),
    ],
    out_specs=[_corespec(128), _corespec(4)],
    out_shape=[_f32((NC, N, 128)), _f32((NC, N, 4))],
)

_tc3 = pl.pallas_call(
    _tc3_body,
    grid=(_GRID,),
    in_specs=[
        _corespec(128), _denspec(), _full((1, 256)), _full((256, 256)),
        _full((1, 256)),
    ],
    out_specs=_rowspec(256),
    out_shape=_f32((N, 256)),
)


def kernel(node_feats, edge_index, edge_attr, np_W, np_b, ep_W, ep_b,
           g1_W, g1_as, g1_ad, g1_b, g2_W, g2_as, g2_ad, g2_b, op_W, op_b):
  i32 = jnp.int32
  loop = jnp.arange(N, dtype=i32)
  src_c = jnp.concatenate(
      [edge_index[0].astype(i32), loop,
       jnp.zeros((EPAD_C - ETOT,), dtype=i32)])
  dst_c = jnp.concatenate(
      [edge_index[1].astype(i32), loop,
       jnp.full((EPAD_C - ETOT,), N, dtype=i32)])
  dst_b = jnp.concatenate(
      [edge_index[1].astype(i32), jnp.full((EPAD_B - E,), N, dtype=i32)])
  ea_pad = jnp.concatenate(
      [edge_attr, jnp.zeros((EPAD_B - E, 16), dtype=jnp.float32)])

  z128 = jnp.zeros((ROWS_PER_TILE, 128), jnp.float32)
  zflat = jnp.zeros((DLEN,), jnp.float32)
  S = (jnp.arange(256)[:, None] // 64 == jnp.arange(4)[None, :]
       ).astype(jnp.float32)
  gsrc = src_c[None, :] + (jnp.arange(NC, dtype=i32) * N)[:, None]

  eacc = _sc_scatter_mean(ea_pad.reshape(EPAD_B * 16),
                          dst_b.reshape(NC * NS * NBLK_B, B), z128)

  h1, t1 = _tc1(node_feats, eacc, np_W, np_b.reshape(1, 256), ep_W,
                ep_b.reshape(1, 256), g1_W, g1_as.reshape(1, 256),
                g1_ad.reshape(1, 256), S)
  t1p = jnp.pad(t1.reshape(NC, 4 * N), ((0, 0), (0, TPAD - 4 * N)))
  w0_1, w1_1, den1 = _sc_gat_alpha(t1p, src_c, dst_c, zflat)
  agg1 = _sc_gat_aggregate(h1.reshape(NC * N, 128), gsrc, w0_1, w1_1, dst_c,
                           z128)

  h2, t2 = _tc2(agg1, den1.reshape(NC, NS, NPAD, 2), g1_b.reshape(1, 256),
                g2_W, g2_as.reshape(1, 256), g2_ad.reshape(1, 256), S)
  t2p = jnp.pad(t2.reshape(NC, 4 * N), ((0, 0), (0, TPAD - 4 * N)))
  w0_2, w1_2, den2 = _sc_gat_alpha(t2p, src_c, dst_c, zflat)
  agg2 = _sc_gat_aggregate(h2.reshape(NC * N, 128), gsrc, w0_2, w1_2, dst_c,
                           z128)

  return _tc3(agg2, den2.reshape(NC, NS, NPAD, 2), g2_b.reshape(1, 256),
              op_W, op_b.reshape(1, 256))
